# baseline (speedup 1.0000x reference)
import dataclasses
import functools

import jax
import jax.numpy as jnp
from jax import lax
from jax.experimental import pallas as pl
from jax.experimental.pallas import tpu as pltpu
from jax.experimental.pallas import tpu_sc as plsc


def _d2h_body(cb_ref, out_ref):
    cb = cb_ref[...]
    out_ref[...] = 0.5 * jnp.sum(cb * cb, axis=0)


def _compute_d2h(codebook):
    k, n = codebook.shape
    bn = 1024
    return pl.pallas_call(
        _d2h_body,
        grid=(n // bn,),
        in_specs=[pl.BlockSpec((k, bn), lambda i: (0, i))],
        out_specs=pl.BlockSpec((bn,), lambda i: (i,)),
        out_shape=jax.ShapeDtypeStruct((n,), jnp.float32),
        compiler_params=pltpu.CompilerParams(
            dimension_semantics=("parallel",)),
    )(codebook)


def _inds_body(x_ref, cb_ref, d2h_ref, inds_ref, *, bn: int):
    x = x_ref[...]
    bm = x.shape[0]
    n = cb_ref.shape[1]
    nchunks = n // bn
    nsub = bn // 128

    norm = jnp.sqrt(jnp.sum(x * x, axis=1, keepdims=True))
    xn = x / jnp.maximum(norm, 1e-12)
    xn_bf = xn.astype(jnp.bfloat16)

    def step(i, carry):
        run_min, run_c = carry
        cb = cb_ref[:, pl.ds(i * bn, bn)]
        d3 = jnp.dot(xn_bf, cb.astype(jnp.bfloat16),
                     preferred_element_type=jnp.float32)
        score = d2h_ref[pl.ds(i * bn, bn)][None, :] - d3
        for s in range(nsub):
            ds_ = score[:, s * 128:(s + 1) * 128]
            upd = ds_ < run_min
            run_min = jnp.where(upd, ds_, run_min)
            run_c = jnp.where(upd, i * nsub + s, run_c)
        return (run_min, run_c)

    init = (jnp.full((bm, 128), jnp.inf, jnp.float32),
            jnp.zeros((bm, 128), jnp.int32))
    run_min, run_c = lax.fori_loop(0, nchunks, step, init, unroll=nchunks)

    lane = lax.broadcasted_iota(jnp.int32, (bm, 128), 1)
    j = run_c * 128 + lane
    m = jnp.min(run_min, axis=1, keepdims=True)
    inds_ref[...] = jnp.min(jnp.where(run_min == m, j, n), axis=1)


def _compute_inds(xf, codebook, d2h, *, bm: int = 512, bn: int = 512):
    mt, k = xf.shape
    n = codebook.shape[1]
    grid = (mt // bm,)
    return pl.pallas_call(
        functools.partial(_inds_body, bn=bn),
        grid=grid,
        in_specs=[
            pl.BlockSpec((bm, k), lambda m: (m, 0)),
            pl.BlockSpec((k, n), lambda m: (0, 0)),
            pl.BlockSpec((n,), lambda m: (0,)),
        ],
        out_specs=pl.BlockSpec((bm,), lambda m: (m,)),
        out_shape=jax.ShapeDtypeStruct((mt,), jnp.int32),
        compiler_params=pltpu.CompilerParams(
            dimension_semantics=("parallel",)),
    )(xf, codebook, d2h)


_ROWS_PER_CHUNK = 4


def _gather_quant(codebook, inds_flat):
    dim, n = codebook.shape
    mt = inds_flat.shape[0]
    info = plsc.get_sparse_core_info()
    nw = info.num_cores * info.num_subcores
    rows_per_w = dim // nw
    rc = _ROWS_PER_CHUNK
    nchunks = rows_per_w // rc
    tsteps = mt // 16

    mesh = plsc.VectorSubcoreMesh(core_axis_name="c", subcore_axis_name="s")
    cp = pltpu.CompilerParams()
    if "needs_layout_passes" in pltpu.CompilerParams.__dataclass_fields__:
        cp = dataclasses.replace(cp, needs_layout_passes=False)

    @functools.partial(
        pl.kernel,
        mesh=mesh,
        compiler_params=cp,
        out_type=jax.ShapeDtypeStruct((dim, mt), jnp.float32),
        scratch_types=[
            pltpu.VMEM((mt,), jnp.int32),
            pltpu.VMEM((rc, n), jnp.float32),
            pltpu.VMEM((rc, mt), jnp.float32),
        ],
    )
    def k(cb_hbm, inds_hbm, out_hbm, inds_v, rows_v, out_v):
        wid = lax.axis_index("s") * info.num_cores + lax.axis_index("c")
        base = wid * rows_per_w
        pltpu.sync_copy(inds_hbm, inds_v)

        def do_chunk(c, _):
            row0 = base + c * rc
            pltpu.sync_copy(cb_hbm.at[pl.ds(row0, rc)], rows_v)

            def tstep(t4, __):
                base_t = t4 * 128
                for u in range(8):
                    off = base_t + u * 16
                    idx = inds_v[pl.ds(off, 16)]
                    for j in range(rc):
                        jv = jnp.full((16,), j, jnp.int32)
                        out_v[j, pl.ds(off, 16)] = plsc.load_gather(
                            rows_v, [jv, idx])
                return __

            lax.fori_loop(0, tsteps // 8, tstep, 0)
            pltpu.sync_copy(out_v, out_hbm.at[pl.ds(row0, rc)])
            return _

        lax.fori_loop(0, nchunks, do_chunk, 0)

    return k(codebook, inds_flat)


def kernel(x, codebook):
    ori_shape = x.shape
    xf = x.reshape(-1, x.shape[-1])
    mt = xf.shape[0]
    nph = 8
    ph = mt // nph
    d2h = _compute_d2h(codebook)
    inds_parts, quant_parts = [], []
    for p in range(nph):
        ip = _compute_inds(xf[p * ph:(p + 1) * ph], codebook, d2h)
        inds_parts.append(ip)
        quant_parts.append(_gather_quant(codebook, ip))
    inds = jnp.concatenate(inds_parts).reshape(ori_shape[:-1])
    quant = jnp.concatenate(quant_parts, axis=1).reshape(
        codebook.shape[0], *ori_shape[:-1])
    return (inds, quant)


# speedup vs baseline: 1.0442x; 1.0442x over previous
import dataclasses
import functools

import jax
import jax.numpy as jnp
from jax import lax
from jax.experimental import pallas as pl
from jax.experimental.pallas import tpu as pltpu
from jax.experimental.pallas import tpu_sc as plsc


def _d2h_body(cb_ref, out_ref):
    cb = cb_ref[...]
    out_ref[...] = 0.5 * jnp.sum(cb * cb, axis=0)


def _compute_d2h(codebook):
    k, n = codebook.shape
    bn = 1024
    return pl.pallas_call(
        _d2h_body,
        grid=(n // bn,),
        in_specs=[pl.BlockSpec((k, bn), lambda i: (0, i))],
        out_specs=pl.BlockSpec((bn,), lambda i: (i,)),
        out_shape=jax.ShapeDtypeStruct((n,), jnp.float32),
        compiler_params=pltpu.CompilerParams(
            dimension_semantics=("parallel",)),
    )(codebook)


def _inds_body(x_ref, cb_ref, d2h_ref, inds_ref, *, bn: int):
    x = x_ref[...]
    bm = x.shape[0]
    n = cb_ref.shape[1]
    nchunks = n // bn
    nsub = bn // 128

    norm = jnp.sqrt(jnp.sum(x * x, axis=1, keepdims=True))
    xn = x / jnp.maximum(norm, 1e-12)
    xn_bf = xn.astype(jnp.bfloat16)

    def step(i, carry):
        run_min, run_c = carry
        cb = cb_ref[:, pl.ds(i * bn, bn)]
        d3 = jnp.dot(xn_bf, cb.astype(jnp.bfloat16),
                     preferred_element_type=jnp.float32)
        score = d2h_ref[pl.ds(i * bn, bn)][None, :] - d3
        for s in range(nsub):
            ds_ = score[:, s * 128:(s + 1) * 128]
            upd = ds_ < run_min
            run_min = jnp.where(upd, ds_, run_min)
            run_c = jnp.where(upd, i * nsub + s, run_c)
        return (run_min, run_c)

    init = (jnp.full((bm, 128), jnp.inf, jnp.float32),
            jnp.zeros((bm, 128), jnp.int32))
    run_min, run_c = lax.fori_loop(0, nchunks, step, init, unroll=nchunks)

    lane = lax.broadcasted_iota(jnp.int32, (bm, 128), 1)
    j = run_c * 128 + lane
    m = jnp.min(run_min, axis=1, keepdims=True)
    inds_ref[...] = jnp.min(jnp.where(run_min == m, j, n), axis=1)


def _compute_inds(xf, codebook, d2h, *, bm: int = 512, bn: int = 512):
    mt, k = xf.shape
    n = codebook.shape[1]
    grid = (mt // bm,)
    return pl.pallas_call(
        functools.partial(_inds_body, bn=bn),
        grid=grid,
        in_specs=[
            pl.BlockSpec((bm, k), lambda m: (m, 0)),
            pl.BlockSpec((k, n), lambda m: (0, 0)),
            pl.BlockSpec((n,), lambda m: (0,)),
        ],
        out_specs=pl.BlockSpec((bm,), lambda m: (m,)),
        out_shape=jax.ShapeDtypeStruct((mt,), jnp.int32),
        compiler_params=pltpu.CompilerParams(
            dimension_semantics=("parallel",)),
    )(xf, codebook, d2h)


_ROWS_PER_CHUNK = 4


def _gather_quant(codebook, inds_flat):
    dim, n = codebook.shape
    mt = inds_flat.shape[0]
    info = plsc.get_sparse_core_info()
    nw = info.num_cores * info.num_subcores
    rows_per_w = dim // nw
    rc = _ROWS_PER_CHUNK
    nchunks = rows_per_w // rc
    tsteps = mt // 16

    mesh = plsc.VectorSubcoreMesh(core_axis_name="c", subcore_axis_name="s")
    cp = pltpu.CompilerParams()
    if "needs_layout_passes" in pltpu.CompilerParams.__dataclass_fields__:
        cp = dataclasses.replace(cp, needs_layout_passes=False)

    @functools.partial(
        pl.kernel,
        mesh=mesh,
        compiler_params=cp,
        out_type=jax.ShapeDtypeStruct((dim, mt), jnp.float32),
        scratch_types=[
            pltpu.VMEM((mt,), jnp.int32),
            pltpu.VMEM((rc, n), jnp.float32),
            pltpu.VMEM((rc, mt), jnp.float32),
        ],
    )
    def k(cb_hbm, inds_hbm, out_hbm, inds_v, rows_v, out_v):
        wid = lax.axis_index("s") * info.num_cores + lax.axis_index("c")
        base = wid * rows_per_w
        pltpu.sync_copy(inds_hbm, inds_v)

        def do_chunk(c, _):
            row0 = base + c * rc
            pltpu.sync_copy(cb_hbm.at[pl.ds(row0, rc)], rows_v)

            def tstep(t4, __):
                base_t = t4 * 128
                for u in range(8):
                    off = base_t + u * 16
                    idx = inds_v[pl.ds(off, 16)]
                    for j in range(rc):
                        jv = jnp.full((16,), j, jnp.int32)
                        out_v[j, pl.ds(off, 16)] = plsc.load_gather(
                            rows_v, [jv, idx])
                return __

            lax.fori_loop(0, tsteps // 8, tstep, 0)
            pltpu.sync_copy(out_v, out_hbm.at[pl.ds(row0, rc)])
            return _

        lax.fori_loop(0, nchunks, do_chunk, 0)

    return k(codebook, inds_flat)


def kernel(x, codebook):
    ori_shape = x.shape
    xf = x.reshape(-1, x.shape[-1])
    mt = xf.shape[0]
    nph = 4
    ph = mt // nph
    d2h = _compute_d2h(codebook)
    inds_parts, quant_parts = [], []
    for p in range(nph):
        ip = _compute_inds(xf[p * ph:(p + 1) * ph], codebook, d2h)
        inds_parts.append(ip)
        quant_parts.append(_gather_quant(codebook, ip))
    inds = jnp.concatenate(inds_parts).reshape(ori_shape[:-1])
    quant = jnp.concatenate(quant_parts, axis=1).reshape(
        codebook.shape[0], *ori_shape[:-1])
    return (inds, quant)


# speedup vs baseline: 1.0626x; 1.0176x over previous
import dataclasses
import functools

import jax
import jax.numpy as jnp
from jax import lax
from jax.experimental import pallas as pl
from jax.experimental.pallas import tpu as pltpu
from jax.experimental.pallas import tpu_sc as plsc


def _d2h_body(cb_ref, out_ref):
    cb = cb_ref[...]
    out_ref[...] = 0.5 * jnp.sum(cb * cb, axis=0)


def _compute_d2h(codebook):
    k, n = codebook.shape
    bn = 1024
    return pl.pallas_call(
        _d2h_body,
        grid=(n // bn,),
        in_specs=[pl.BlockSpec((k, bn), lambda i: (0, i))],
        out_specs=pl.BlockSpec((bn,), lambda i: (i,)),
        out_shape=jax.ShapeDtypeStruct((n,), jnp.float32),
        compiler_params=pltpu.CompilerParams(
            dimension_semantics=("parallel",)),
    )(codebook)


def _inds_body(x_ref, cb_ref, d2h_ref, inds_ref, *, bn: int):
    x = x_ref[...]
    bm = x.shape[0]
    n = cb_ref.shape[1]
    nchunks = n // bn
    nsub = bn // 128

    norm = jnp.sqrt(jnp.sum(x * x, axis=1, keepdims=True))
    xn = x / jnp.maximum(norm, 1e-12)
    xn_bf = xn.astype(jnp.bfloat16)

    def step(i, carry):
        run_min, run_c = carry
        cb = cb_ref[:, pl.ds(i * bn, bn)]
        d3 = jnp.dot(xn_bf, cb.astype(jnp.bfloat16),
                     preferred_element_type=jnp.float32)
        score = d2h_ref[pl.ds(i * bn, bn)][None, :] - d3
        for s in range(nsub):
            ds_ = score[:, s * 128:(s + 1) * 128]
            upd = ds_ < run_min
            run_min = jnp.where(upd, ds_, run_min)
            run_c = jnp.where(upd, i * nsub + s, run_c)
        return (run_min, run_c)

    init = (jnp.full((bm, 128), jnp.inf, jnp.float32),
            jnp.zeros((bm, 128), jnp.int32))
    run_min, run_c = lax.fori_loop(0, nchunks, step, init, unroll=nchunks)

    lane = lax.broadcasted_iota(jnp.int32, (bm, 128), 1)
    j = run_c * 128 + lane
    m = jnp.min(run_min, axis=1, keepdims=True)
    inds_ref[...] = jnp.min(jnp.where(run_min == m, j, n), axis=1)


def _compute_inds(xf, codebook, d2h, *, bm: int = 512, bn: int = 512):
    mt, k = xf.shape
    n = codebook.shape[1]
    grid = (mt // bm,)
    return pl.pallas_call(
        functools.partial(_inds_body, bn=bn),
        grid=grid,
        in_specs=[
            pl.BlockSpec((bm, k), lambda m: (m, 0)),
            pl.BlockSpec((k, n), lambda m: (0, 0)),
            pl.BlockSpec((n,), lambda m: (0,)),
        ],
        out_specs=pl.BlockSpec((bm,), lambda m: (m,)),
        out_shape=jax.ShapeDtypeStruct((mt,), jnp.int32),
        compiler_params=pltpu.CompilerParams(
            dimension_semantics=("parallel",)),
    )(xf, codebook, d2h)


_ROWS_PER_CHUNK = 8


def _gather_quant(codebook, inds_flat):
    dim, n = codebook.shape
    mt = inds_flat.shape[0]
    info = plsc.get_sparse_core_info()
    nw = info.num_cores * info.num_subcores
    rows_per_w = dim // nw
    rc = _ROWS_PER_CHUNK
    nchunks = rows_per_w // rc
    tsteps = mt // 16

    mesh = plsc.VectorSubcoreMesh(core_axis_name="c", subcore_axis_name="s")
    cp = pltpu.CompilerParams()
    if "needs_layout_passes" in pltpu.CompilerParams.__dataclass_fields__:
        cp = dataclasses.replace(cp, needs_layout_passes=False)

    @functools.partial(
        pl.kernel,
        mesh=mesh,
        compiler_params=cp,
        out_type=jax.ShapeDtypeStruct((dim, mt), jnp.float32),
        scratch_types=[
            pltpu.VMEM((mt,), jnp.int32),
            pltpu.VMEM((rc, n), jnp.float32),
            pltpu.VMEM((rc, mt), jnp.float32),
        ],
    )
    def k(cb_hbm, inds_hbm, out_hbm, inds_v, rows_v, out_v):
        wid = lax.axis_index("s") * info.num_cores + lax.axis_index("c")
        base = wid * rows_per_w
        pltpu.sync_copy(inds_hbm, inds_v)

        def do_chunk(c, _):
            row0 = base + c * rc
            pltpu.sync_copy(cb_hbm.at[pl.ds(row0, rc)], rows_v)

            def tstep(t4, __):
                base_t = t4 * 128
                for u in range(8):
                    off = base_t + u * 16
                    idx = inds_v[pl.ds(off, 16)]
                    for j in range(rc):
                        jv = jnp.full((16,), j, jnp.int32)
                        out_v[j, pl.ds(off, 16)] = plsc.load_gather(
                            rows_v, [jv, idx])
                return __

            lax.fori_loop(0, tsteps // 8, tstep, 0)
            pltpu.sync_copy(out_v, out_hbm.at[pl.ds(row0, rc)])
            return _

        lax.fori_loop(0, nchunks, do_chunk, 0)

    return k(codebook, inds_flat)


def kernel(x, codebook):
    ori_shape = x.shape
    xf = x.reshape(-1, x.shape[-1])
    mt = xf.shape[0]
    nph = 4
    ph = mt // nph
    d2h = _compute_d2h(codebook)
    inds_parts, quant_parts = [], []
    for p in range(nph):
        ip = _compute_inds(xf[p * ph:(p + 1) * ph], codebook, d2h)
        inds_parts.append(ip)
        quant_parts.append(_gather_quant(codebook, ip))
    inds = jnp.concatenate(inds_parts).reshape(ori_shape[:-1])
    quant = jnp.concatenate(quant_parts, axis=1).reshape(
        codebook.shape[0], *ori_shape[:-1])
    return (inds, quant)
